# baseline (speedup 1.0000x reference)
import functools

import jax
import jax.numpy as jnp
from jax import lax
from jax.experimental import pallas as pl
from jax.experimental.pallas import tpu as pltpu
from jax.experimental.pallas import tpu_sc as plsc

N_NODES = 10000
N_PAD = 10112
N_EDGES = 320000
D = 128
DG = 144
NC = 2
NS = 16
NW = NC * NS
K = 80
EPT = N_EDGES // NW
CPT = EPT // K
SB = 32
ROWS_PT = N_PAD // NS
NBUF = 3

_BLOCKS = ((0, 32, 0), (32, 32, 32), (64, 32, 64), (96, 29, 93))

_sc_mesh = plsc.VectorSubcoreMesh(core_axis_name="c", subcore_axis_name="s")


@functools.partial(
    pl.kernel,
    out_type=jax.ShapeDtypeStruct((NC, N_PAD, DG), jnp.float32),
    mesh=_sc_mesh,
    scratch_types=[
        pltpu.VMEM((SB, K), jnp.int32),
        pltpu.VMEM((SB, K), jnp.int32),
        pltpu.VMEM((K, DG), jnp.float32),
        pltpu.VMEM((K, DG), jnp.float32),
        pltpu.VMEM((K, DG), jnp.float32),
        pltpu.VMEM_SHARED((N_PAD, DG), jnp.float32),
        pltpu.SemaphoreType.DMA,
        pltpu.SemaphoreType.DMA,
        pltpu.SemaphoreType.DMA,
        pltpu.SemaphoreType.DMA,
        pltpu.SemaphoreType.DMA,
        pltpu.SemaphoreType.DMA,
    ],
    compiler_params=pltpu.CompilerParams(use_tc_tiling_on_sc=False),
)
def _sc_aggregate(x_hbm, src_hbm, dst_hbm, feat_out,
                  src_v, dst_v, rowsA, rowsB, rowsC,
                  feat_acc, sgA, sgB, sgC, sfA, sfB, sfC):
    cid = lax.axis_index("c")
    sid = lax.axis_index("s")
    tid = cid * NS + sid

    ROWS = (rowsA, rowsB, rowsC)
    SG = (sgA, sgB, sgC)
    SF = (sfA, sfB, sfC)

    zero16 = jnp.zeros((16,), jnp.float32)

    def rows_zero(i, _):
        rowsA[i // (DG // 16), pl.ds((i % (DG // 16)) * 16, 16)] = zero16
        return 0
    lax.fori_loop(0, K * (DG // 16), rows_zero, 0)

    for b in range(ROWS_PT // K):
        pltpu.sync_copy(rowsA, feat_acc.at[pl.ds(sid * ROWS_PT + b * K, K)])
    _rem = ROWS_PT % K
    if _rem:
        pltpu.sync_copy(
            rowsA.at[pl.ds(0, _rem)],
            feat_acc.at[pl.ds(sid * ROWS_PT + (ROWS_PT // K) * K, _rem)])

    def stage(soff):
        pltpu.sync_copy(src_hbm.at[tid].at[pl.ds(soff, SB)], src_v)
        pltpu.sync_copy(dst_hbm.at[tid].at[pl.ds(soff, SB)], dst_v)

    def gather(j, soff, b):
        pltpu.async_copy(x_hbm.at[src_v.at[j - soff]], ROWS[b], SG[b])

    def wait_gather(j, soff, b):
        pltpu.make_async_copy(x_hbm.at[src_v.at[j - soff]], ROWS[b], SG[b]).wait()

    def scatter(j, soff, b):
        pltpu.async_copy(ROWS[b], feat_acc.at[dst_v.at[j - soff]], SF[b],
                         add=True)

    def wait_scatter(j, soff, b):
        pltpu.make_async_copy(ROWS[b], feat_acc.at[dst_v.at[j - soff]],
                              SF[b]).wait()

    def run_block(start, count, soff, first):
        end = start + count
        b0 = start % NBUF
        for u in range(NBUF):
            gather(start + u, soff, (b0 + u) % NBUF)
        if first:
            plsc.subcore_barrier()
        wait_gather(start, soff, b0)
        scatter(start, soff, b0)
        n3 = (count - 1) // NBUF
        tail = (count - 1) % NBUF

        def body(m, _):
            for u in range(NBUF):
                j = start + 1 + NBUF * m + u
                b = (b0 + 1 + u) % NBUF
                wait_scatter(j - 1, soff, (b + 2) % NBUF)

                @pl.when(j + 2 < end)
                def _():
                    gather(j + 2, soff, (b + 2) % NBUF)

                wait_gather(j, soff, b)
                scatter(j, soff, b)
            return 0
        lax.fori_loop(0, n3, body, 0)
        for u in range(tail):
            j = start + 1 + NBUF * n3 + u
            b = (b0 + (j - start)) % NBUF
            wait_scatter(j - 1, soff, (b + 2) % NBUF)
            if j + 2 < end:
                gather(j + 2, soff, (b + 2) % NBUF)
            wait_gather(j, soff, b)
            scatter(j, soff, b)
        wait_scatter(end - 1, soff, (b0 + count - 1) % NBUF)

    for bi, (start, count, soff) in enumerate(_BLOCKS):
        stage(soff)
        run_block(start, count, soff, first=(bi == 0))

    plsc.subcore_barrier()

    pltpu.sync_copy(feat_acc.at[pl.ds(sid * ROWS_PT, ROWS_PT)],
                    feat_out.at[cid].at[pl.ds(sid * ROWS_PT, ROWS_PT)])


def _tc_body(x_ref, f_ref, wst_ref, bs_ref, wnt_ref, bn_ref, o_ref):
    xb = x_ref[...]
    f = f_ref[0] + f_ref[1]
    cnt = f[:, D:D + 1]
    self_t = jnp.dot(xb, wst_ref[...], preferred_element_type=jnp.float32)
    self_t = self_t + bs_ref[...]
    agg = jnp.dot(f[:, :D], wnt_ref[...], preferred_element_type=jnp.float32)
    neigh = (agg + cnt * bn_ref[...]) / jnp.maximum(cnt, 1.0)
    o_ref[...] = jnp.maximum(self_t + neigh, 0.0)


_R = 1000

_tc_combine = pl.pallas_call(
    _tc_body,
    out_shape=jax.ShapeDtypeStruct((N_NODES, D), jnp.float32),
    grid=(N_NODES // _R,),
    in_specs=[
        pl.BlockSpec((_R, D), lambda i: (i, 0)),
        pl.BlockSpec((NC, _R, DG), lambda i: (0, i, 0)),
        pl.BlockSpec((D, D), lambda i: (0, 0)),
        pl.BlockSpec((1, D), lambda i: (0, 0)),
        pl.BlockSpec((D, D), lambda i: (0, 0)),
        pl.BlockSpec((1, D), lambda i: (0, 0)),
    ],
    out_specs=pl.BlockSpec((_R, D), lambda i: (i, 0)),
)


def kernel(x, edge_index, W_self, b_self, W_neighbor, b_neighbor):
    ei = edge_index.astype(jnp.int32)
    src3d = ei[0].reshape(NW, CPT, K)
    dst3d = ei[1].reshape(NW, CPT, K)
    pad = jnp.concatenate(
        [jnp.ones((N_NODES, 1), jnp.float32),
         jnp.zeros((N_NODES, DG - D - 1), jnp.float32)], axis=1)
    x144 = jnp.concatenate([x, pad], axis=1)
    feat_par = _sc_aggregate(x144, src3d, dst3d)
    return _tc_combine(x, feat_par,
                       W_self.T, b_self.reshape(1, D),
                       W_neighbor.T, b_neighbor.reshape(1, D))


# speedup vs baseline: 1.2960x; 1.2960x over previous
import functools

import jax
import jax.numpy as jnp
from jax import lax
from jax.experimental import pallas as pl
from jax.experimental.pallas import tpu as pltpu
from jax.experimental.pallas import tpu_sc as plsc

N_NODES = 10000
N_PAD = 10240
N_EDGES = 320000
D = 128
NC = 2
NS = 16
NW = NC * NS
K = 80
EPT = N_EDGES // NW
CPT = EPT // K
SB = 32
ROWS_PT = N_PAD // NS
CW = 16
NBUF = 3

_BLOCKS = ((0, 32, 0), (32, 32, 32), (64, 32, 64), (96, 29, 93))

_sc_mesh = plsc.VectorSubcoreMesh(core_axis_name="c", subcore_axis_name="s")


@functools.partial(
    pl.kernel,
    out_type=(
        jax.ShapeDtypeStruct((NC, N_PAD, D), jnp.float32),
        jax.ShapeDtypeStruct((NC, N_PAD, CW), jnp.float32),
    ),
    mesh=_sc_mesh,
    scratch_types=[
        pltpu.VMEM((SB, K), jnp.int32),
        pltpu.VMEM((SB, K), jnp.int32),
        pltpu.VMEM((K, D), jnp.float32),
        pltpu.VMEM((K, D), jnp.float32),
        pltpu.VMEM((K, D), jnp.float32),
        pltpu.VMEM((K, CW), jnp.float32),
        pltpu.VMEM_SHARED((N_PAD, D), jnp.float32),
        pltpu.VMEM_SHARED((N_PAD, CW), jnp.float32),
        pltpu.SemaphoreType.DMA,
        pltpu.SemaphoreType.DMA,
        pltpu.SemaphoreType.DMA,
        pltpu.SemaphoreType.DMA,
        pltpu.SemaphoreType.DMA,
        pltpu.SemaphoreType.DMA,
        pltpu.SemaphoreType.DMA,
        pltpu.SemaphoreType.DMA,
        pltpu.SemaphoreType.DMA,
    ],
    compiler_params=pltpu.CompilerParams(use_tc_tiling_on_sc=False),
)
def _sc_aggregate(x_hbm, ei_hbm, feat_out, cnt_out,
                  src_v, dst_v, rowsA, rowsB, rowsC, ones_v,
                  feat_acc, cnt_acc, sgA, sgB, sgC, sfA, sfB, sfC,
                  scA, scB, scC):
    cid = lax.axis_index("c")
    sid = lax.axis_index("s")
    tid = cid * NS + sid

    ROWS = (rowsA, rowsB, rowsC)
    SG = (sgA, sgB, sgC)
    SF = (sfA, sfB, sfC)
    SC = (scA, scB, scC)

    zero16 = jnp.zeros((16,), jnp.float32)
    onehot = jnp.where(lax.iota(jnp.int32, 16) == 0, 1.0, 0.0)

    def rows_zero(i, _):
        rowsA[i // (D // 16), pl.ds((i % (D // 16)) * 16, 16)] = zero16
        return 0
    lax.fori_loop(0, K * (D // 16), rows_zero, 0)

    def ones_zero(i, _):
        ones_v[i, pl.ds(0, CW)] = zero16
        return 0
    lax.fori_loop(0, K, ones_zero, 0)

    for b in range(ROWS_PT // K):
        pltpu.sync_copy(rowsA, feat_acc.at[pl.ds(sid * ROWS_PT + b * K, K)])
        pltpu.sync_copy(ones_v, cnt_acc.at[pl.ds(sid * ROWS_PT + b * K, K)])

    def ones_fill(i, _):
        ones_v[i, pl.ds(0, CW)] = onehot
        return 0
    lax.fori_loop(0, K, ones_fill, 0)

    def stage(soff):
        pltpu.sync_copy(ei_hbm.at[0].at[tid].at[pl.ds(soff, SB)], src_v)
        pltpu.sync_copy(ei_hbm.at[1].at[tid].at[pl.ds(soff, SB)], dst_v)

    def gather(j, soff, b):
        pltpu.async_copy(x_hbm.at[src_v.at[j - soff]], ROWS[b], SG[b])

    def wait_gather(j, soff, b):
        pltpu.make_async_copy(x_hbm.at[src_v.at[j - soff]], ROWS[b], SG[b]).wait()

    def scatters(j, soff, b):
        pltpu.async_copy(ROWS[b], feat_acc.at[dst_v.at[j - soff]], SF[b],
                         add=True)
        pltpu.async_copy(ones_v, cnt_acc.at[dst_v.at[j - soff]], SC[b],
                         add=True)

    def wait_scatters(j, soff, b):
        pltpu.make_async_copy(ROWS[b], feat_acc.at[dst_v.at[j - soff]],
                              SF[b]).wait()
        pltpu.make_async_copy(ones_v, cnt_acc.at[dst_v.at[j - soff]],
                              SC[b]).wait()

    def run_block(start, count, soff, first):
        end = start + count
        b0 = start % NBUF
        for u in range(NBUF):
            gather(start + u, soff, (b0 + u) % NBUF)
        if first:
            plsc.subcore_barrier()
        wait_gather(start, soff, b0)
        scatters(start, soff, b0)
        n3 = (count - 1) // NBUF
        tail = (count - 1) % NBUF

        def body(m, _):
            for u in range(NBUF):
                j = start + 1 + NBUF * m + u
                b = (b0 + 1 + u) % NBUF
                wait_scatters(j - 1, soff, (b + 2) % NBUF)

                @pl.when(j + 2 < end)
                def _():
                    gather(j + 2, soff, (b + 2) % NBUF)

                wait_gather(j, soff, b)
                scatters(j, soff, b)
            return 0
        lax.fori_loop(0, n3, body, 0)
        for u in range(tail):
            j = start + 1 + NBUF * n3 + u
            b = (b0 + (j - start)) % NBUF
            wait_scatters(j - 1, soff, (b + 2) % NBUF)
            if j + 2 < end:
                gather(j + 2, soff, (b + 2) % NBUF)
            wait_gather(j, soff, b)
            scatters(j, soff, b)
        wait_scatters(end - 1, soff, (b0 + count - 1) % NBUF)

    for bi, (start, count, soff) in enumerate(_BLOCKS):
        stage(soff)
        run_block(start, count, soff, first=(bi == 0))

    plsc.subcore_barrier()

    pltpu.sync_copy(feat_acc.at[pl.ds(sid * ROWS_PT, ROWS_PT)],
                    feat_out.at[cid].at[pl.ds(sid * ROWS_PT, ROWS_PT)])
    pltpu.sync_copy(cnt_acc.at[pl.ds(sid * ROWS_PT, ROWS_PT)],
                    cnt_out.at[cid].at[pl.ds(sid * ROWS_PT, ROWS_PT)])


def _tc_body(x_ref, f_ref, c_ref, wst_ref, bs_ref, wnt_ref, bn_ref, o_ref):
    xb = x_ref[...]
    f = f_ref[0] + f_ref[1]
    c = c_ref[0] + c_ref[1]
    cnt = c[:, 0:1]
    self_t = jnp.dot(xb, wst_ref[...], preferred_element_type=jnp.float32)
    self_t = self_t + bs_ref[...]
    agg = jnp.dot(f, wnt_ref[...], preferred_element_type=jnp.float32)
    neigh = (agg + cnt * bn_ref[...]) / jnp.maximum(cnt, 1.0)
    o_ref[...] = jnp.maximum(self_t + neigh, 0.0)


_R = 2000

_tc_combine = pl.pallas_call(
    _tc_body,
    out_shape=jax.ShapeDtypeStruct((N_NODES, D), jnp.float32),
    grid=(N_NODES // _R,),
    in_specs=[
        pl.BlockSpec((_R, D), lambda i: (i, 0)),
        pl.BlockSpec((NC, _R, D), lambda i: (0, i, 0)),
        pl.BlockSpec((NC, _R, CW), lambda i: (0, i, 0)),
        pl.BlockSpec((D, D), lambda i: (0, 0)),
        pl.BlockSpec((1, D), lambda i: (0, 0)),
        pl.BlockSpec((D, D), lambda i: (0, 0)),
        pl.BlockSpec((1, D), lambda i: (0, 0)),
    ],
    out_specs=pl.BlockSpec((_R, D), lambda i: (i, 0)),
)


def kernel(x, edge_index, W_self, b_self, W_neighbor, b_neighbor):
    ei4d = edge_index.astype(jnp.int32).reshape(2, NW, CPT, K)
    feat_par, cnt_par = _sc_aggregate(x, ei4d)
    return _tc_combine(x, feat_par, cnt_par,
                       W_self.T, b_self.reshape(1, D),
                       W_neighbor.T, b_neighbor.reshape(1, D))
